# baseline (speedup 1.0000x reference)
import jax
import jax.numpy as jnp
from jax import lax
from jax.experimental import pallas as pl
from jax.experimental.pallas import tpu as pltpu
from jax.experimental.pallas import tpu_sc as plsc

_NC = 2
_NS = 16
_NW = _NC * _NS
_L = 16

_VOCAB = 1000000
_EMBD = 64
_BATCH = 4096
_SEQ = 200

_ROWS_PER_W = _BATCH // _NW
_CHUNKS = (104, 96)
_COLS = (0, 104)
_STEPS = _ROWS_PER_W * 2
_NBUF = 4

_MASK_OFFS = tuple(range(0, _SEQ - _L, _L)) + (_SEQ - _L,)


def _emb_kernel(table_hbm, x_hbm, emb_hbm, mask_hbm,
                idx_v, mask_v, rows, gsems, psems):
  wid = lax.axis_index("s") * _NC + lax.axis_index("c")
  row0 = wid * _ROWS_PER_W

  pltpu.sync_copy(x_hbm.at[pl.ds(row0, _ROWS_PER_W)], idx_v)

  def gather_ref(j, b):
    r = lax.div(j, 2) if isinstance(j, jax.Array) else j // 2
    h = b & 1
    return table_hbm.at[idx_v.at[r, pl.ds(_COLS[h], _CHUNKS[h])]]

  def out_ref(j, b):
    r = lax.div(j, 2) if isinstance(j, jax.Array) else j // 2
    h = b & 1
    return emb_hbm.at[row0 + r, pl.ds(_COLS[h], _CHUNKS[h])]

  def start_gather(j, b, slot):
    pltpu.async_copy(gather_ref(j, b), rows[slot], gsems[slot])

  def wait_gather(j, b, slot):
    pltpu.make_async_copy(gather_ref(j, b), rows[slot], gsems[slot]).wait()

  def start_put(j, b, slot):
    pltpu.async_copy(rows[slot], out_ref(j, b), psems[slot])

  def wait_put(j, b, slot):
    pltpu.make_async_copy(rows[slot], out_ref(j, b), psems[slot]).wait()

  def mask_row(r):
    for o in _MASK_OFFS:
      v = idx_v[r, pl.ds(o, _L)]
      mask_v[r, pl.ds(o, _L)] = jnp.minimum(v, 1).astype(jnp.float32)

  def step(j, b, *, head, tail):
    if not head:
      wait_put(j - 2, b, (b + 2) % _NBUF)
    if not tail:
      start_gather(j + 2, b, (b + 2) % _NBUF)
    wait_gather(j, b, b)
    start_put(j, b, b)
    if b & 1:
      mask_row(lax.div(j, 2) if isinstance(j, jax.Array) else j // 2)

  start_gather(0, 0, 0)
  start_gather(1, 1, 1)
  for b in range(_NBUF):
    step(b, b, head=(b < 2), tail=False)

  def body(jj, _):
    for b in range(_NBUF):
      step(jj * _NBUF + b, b, head=False, tail=False)
    return 0

  lax.fori_loop(1, _STEPS // _NBUF - 1, body, 0)

  last = _STEPS - _NBUF
  for b in range(_NBUF):
    step(last + b, b, head=False, tail=(b >= 2))
  wait_put(_STEPS - 2, 2, 2)
  wait_put(_STEPS - 1, 3, 3)

  pltpu.sync_copy(mask_v, mask_hbm.at[pl.ds(row0, _ROWS_PER_W)])


@jax.jit
def kernel(x, table):
  mesh = plsc.VectorSubcoreMesh(core_axis_name="c", subcore_axis_name="s")
  emb, mask = pl.kernel(
      _emb_kernel,
      out_type=(
          jax.ShapeDtypeStruct((_BATCH, _SEQ, _EMBD), jnp.float32),
          jax.ShapeDtypeStruct((_BATCH, _SEQ), jnp.float32),
      ),
      mesh=mesh,
      scratch_types=(
          pltpu.VMEM((_ROWS_PER_W, _SEQ), jnp.int32),
          pltpu.VMEM((_ROWS_PER_W, _SEQ), jnp.float32),
          tuple(pltpu.VMEM((_CHUNKS[b & 1], _EMBD), jnp.float32)
                for b in range(_NBUF)),
          tuple(pltpu.SemaphoreType.DMA for _ in range(_NBUF)),
          tuple(pltpu.SemaphoreType.DMA for _ in range(_NBUF)),
      ),
      compiler_params=pltpu.CompilerParams(use_tc_tiling_on_sc=False),
  )(table, x.astype(jnp.int32))
  return emb, mask


# speedup vs baseline: 1.0036x; 1.0036x over previous
import jax
import jax.numpy as jnp
from jax import lax
from jax.experimental import pallas as pl
from jax.experimental.pallas import tpu as pltpu
from jax.experimental.pallas import tpu_sc as plsc

_NC = 2
_NS = 16
_NW = _NC * _NS
_L = 16

_VOCAB = 1000000
_EMBD = 64
_BATCH = 4096
_SEQ = 200

_DR = 2 * _SEQ
_DR_PER_W = _BATCH // 2 // _NW
_OFFS = (0, 128, 256, 352)
_CHS = (128, 128, 96, 48)
_NBUF = 4
_STEPS = _DR_PER_W * _NBUF


def _emb_kernel(t_hbm, x2_hbm, emb2_hbm, mask2_hbm,
                idx_v, mask_v, gbufs, gsems, psems):
  wid = lax.axis_index("s") * _NC + lax.axis_index("c")
  dr0 = wid * _DR_PER_W

  pltpu.sync_copy(x2_hbm.at[pl.ds(dr0, _DR_PER_W)], idx_v)

  def g_refs(j, c):
    k = lax.div(j, _NBUF)
    return t_hbm.at[idx_v.at[k, pl.ds(_OFFS[c], _CHS[c])]], gbufs[c]

  def start_gather(j, c):
    src, dst = g_refs(j, c)
    pltpu.async_copy(src, dst, gsems[c])

  def wait_gather(j, c):
    src, dst = g_refs(j, c)
    pltpu.make_async_copy(src, dst, gsems[c]).wait()

  def out_ref(j, c):
    k = lax.div(j, _NBUF)
    r0 = (dr0 + k) * _DR + _OFFS[c]
    return emb2_hbm.at[pl.ds(r0, _CHS[c])]

  def start_put(j, c):
    pltpu.async_copy(gbufs[c], out_ref(j, c), psems[c])

  def wait_put(j, c):
    pltpu.make_async_copy(gbufs[c], out_ref(j, c), psems[c]).wait()

  def mask_row(k):
    for o in range(0, _DR, _L):
      v = idx_v[k, pl.ds(o, _L)]
      mask_v[k, pl.ds(o, _L)] = jnp.minimum(v, 1).astype(jnp.float32)

  def step(j, c, *, head, tail):
    c2 = (c + 2) % _NBUF
    if not head:
      wait_put(j - 2, c2)
    if not tail:
      start_gather(j + 2, c2)
    wait_gather(j, c)
    start_put(j, c)
    if c == 0:
      mask_row(lax.div(j, _NBUF))

  start_gather(0, 0)
  start_gather(1, 1)
  for c in range(_NBUF):
    step(c, c, head=(c < 2), tail=False)

  def body(u, _):
    for c in range(_NBUF):
      step(u * _NBUF + c, c, head=False, tail=False)
    return 0

  lax.fori_loop(1, _STEPS // _NBUF - 1, body, 0)

  last = _STEPS - _NBUF
  for c in range(_NBUF):
    step(last + c, c, head=False, tail=(c >= 2))
  wait_put(_STEPS - 2, 2)
  wait_put(_STEPS - 1, 3)

  pltpu.sync_copy(mask_v, mask2_hbm.at[pl.ds(dr0, _DR_PER_W)])


@jax.jit
def kernel(x, table):
  x2 = x.reshape(_BATCH // 2, _DR).astype(jnp.int32)
  mesh = plsc.VectorSubcoreMesh(core_axis_name="c", subcore_axis_name="s")
  emb2, mask2 = pl.kernel(
      _emb_kernel,
      out_type=(
          jax.ShapeDtypeStruct((_BATCH * _SEQ, _EMBD), jnp.float32),
          jax.ShapeDtypeStruct((_BATCH // 2, _DR), jnp.float32),
      ),
      mesh=mesh,
      scratch_types=(
          pltpu.VMEM((_DR_PER_W, _DR), jnp.int32),
          pltpu.VMEM((_DR_PER_W, _DR), jnp.float32),
          tuple(pltpu.VMEM((_CHS[c], _EMBD), jnp.float32)
                for c in range(_NBUF)),
          tuple(pltpu.SemaphoreType.DMA for _ in range(_NBUF)),
          tuple(pltpu.SemaphoreType.DMA for _ in range(_NBUF)),
      ),
      compiler_params=pltpu.CompilerParams(use_tc_tiling_on_sc=False,
                                           needs_layout_passes=False),
  )(table, x2)
  return emb2.reshape(_BATCH, _SEQ, _EMBD), mask2.reshape(_BATCH, _SEQ)
